# Initial kernel scaffold; baseline (speedup 1.0000x reference)
import jax
import jax.numpy as jnp
from jax.experimental import pallas as pl


def kernel(b, s, buyer_factors, seller_factors):
    raise NotImplementedError("write your pallas kernel here")


# speedup vs baseline: 2.3623x; 2.3623x over previous
import functools

import jax
import jax.numpy as jnp
from jax import lax
from jax.experimental import pallas as pl
from jax.experimental.pallas import tpu as pltpu
from jax.experimental.pallas import tpu_sc as plsc

B = 4096
H = 50
F = 128
NW = 32
ROWS_PER_W = B // NW
CHUNK_ROWS = 4
PAIR_COLS = 100
VROWS_PER_W = ROWS_PER_W * H // PAIR_COLS
N_CHUNKS = ROWS_PER_W // CHUNK_ROWS
L = 16
NVREG = F // L


def _lane_gather(v, idx):
    dn = lax.GatherDimensionNumbers(
        offset_dims=(), collapsed_slice_dims=(0,), start_index_map=(0,))
    return lax.gather(v, idx[:, None], dn, slice_sizes=(1,),
                      mode=lax.GatherScatterMode.PROMISE_IN_BOUNDS)


def _body(bidx, sidx, btab, stab, out, idxb_v, idxs_v, brow_v, srow_v,
          out_v, sem):
    wid = lax.axis_index("s") * 2 + lax.axis_index("c")
    ibase = wid * VROWS_PER_W

    def chunk_body(c, carry):
        r0 = ibase + 2 * c
        pltpu.sync_copy(bidx.at[pl.ds(r0, 2)], idxb_v)
        pltpu.sync_copy(sidx.at[pl.ds(r0, 2)], idxs_v)
        cps = []
        for h in range(2):
            cps.append(pltpu.async_copy(btab.at[idxb_v.at[h]], brow_v.at[h], sem))
            cps.append(pltpu.async_copy(stab.at[idxs_v.at[h]], srow_v.at[h], sem))
        for cp in cps:
            cp.wait()
        for r in range(CHUNK_ROWS):
            half = r // 2
            off = (r % 2) * H

            def jbody(j, acc, half=half, off=off):
                return tuple(
                    acc[k]
                    + brow_v[half, off + j, k]
                    * srow_v[half, off + j, k]
                    for k in range(NVREG)
                )

            acc0 = tuple(jnp.zeros((L,), jnp.float32) for _ in range(NVREG))
            acc = lax.fori_loop(0, H, jbody, acc0)
            v = ((acc[0] + acc[1]) + (acc[2] + acc[3])) + (
                (acc[4] + acc[5]) + (acc[6] + acc[7]))
            lanes = lax.iota(jnp.int32, L)
            for sh in (1, 2, 4, 8):
                v = v + _lane_gather(v, lanes ^ sh)
            pos = CHUNK_ROWS * c + r
            plsc.store_scatter(
                out_v,
                [jnp.full((L,), pos, jnp.int32)],
                v,
                mask=lanes == 0,
            )
        return carry

    lax.fori_loop(0, N_CHUNKS, chunk_body, 0)
    pltpu.sync_copy(out_v, out.at[pl.ds(wid * ROWS_PER_W, ROWS_PER_W)])


@jax.jit
def _mf(bidx, sidx, btab, stab):
    mesh = plsc.VectorSubcoreMesh(core_axis_name="c", subcore_axis_name="s")
    return pl.kernel(
        _body,
        out_type=jax.ShapeDtypeStruct((B,), jnp.float32),
        mesh=mesh,
        scratch_types=[
            pltpu.VMEM((2, PAIR_COLS), jnp.int32),
            pltpu.VMEM((2, PAIR_COLS), jnp.int32),
            pltpu.VMEM((2, PAIR_COLS, NVREG, L), jnp.float32),
            pltpu.VMEM((2, PAIR_COLS, NVREG, L), jnp.float32),
            pltpu.VMEM((ROWS_PER_W,), jnp.float32),
            pltpu.SemaphoreType.DMA,
        ],
        compiler_params=pltpu.CompilerParams(needs_layout_passes=False, use_tc_tiling_on_sc=False),
    )(bidx, sidx, btab, stab)


def kernel(b, s, buyer_factors, seller_factors):
    bidx = b.reshape(B * H // PAIR_COLS, PAIR_COLS)
    sidx = s.reshape(B * H // PAIR_COLS, PAIR_COLS)
    btab = buyer_factors.reshape(-1, NVREG, L)
    stab = seller_factors.reshape(-1, NVREG, L)
    return _mf(bidx, sidx, btab, stab)


# speedup vs baseline: 9.9348x; 4.2055x over previous
import jax
import jax.numpy as jnp
from jax import lax
from jax.experimental import pallas as pl
from jax.experimental.pallas import tpu as pltpu
from jax.experimental.pallas import tpu_sc as plsc

B = 4096
H = 50
F = 128
NW = 32
ROWS_PER_W = B // NW
CHUNK_ROWS = 4
N_CHUNKS = ROWS_PER_W // CHUNK_ROWS
L = 16
NVREG = F // L


def _lane_gather(v, idx):
    dn = lax.GatherDimensionNumbers(
        offset_dims=(), collapsed_slice_dims=(0,), start_index_map=(0,))
    return lax.gather(v, idx[:, None], dn, slice_sizes=(1,),
                      mode=lax.GatherScatterMode.PROMISE_IN_BOUNDS)


def _body(bidx, sidx, btab, stab, out, idxb_v, idxs_v, brow_v, srow_v,
          out_v, sem):
    wid = lax.axis_index("s") * 2 + lax.axis_index("c")
    obase = wid * ROWS_PER_W

    def chunk_body(c, carry):
        r0 = obase + CHUNK_ROWS * c
        pltpu.sync_copy(bidx.at[pl.ds(r0, CHUNK_ROWS)], idxb_v)
        pltpu.sync_copy(sidx.at[pl.ds(r0, CHUNK_ROWS)], idxs_v)
        cps = []
        for r in range(CHUNK_ROWS):
            cps.append(pltpu.async_copy(btab.at[idxb_v.at[r]], brow_v.at[r], sem))
            cps.append(pltpu.async_copy(stab.at[idxs_v.at[r]], srow_v.at[r], sem))
        for cp in cps:
            cp.wait()
        for r in range(CHUNK_ROWS):
            def jbody(j, acc, r=r):
                return tuple(
                    acc[k]
                    + brow_v[r, j, pl.ds(k * L, L)]
                    * srow_v[r, j, pl.ds(k * L, L)]
                    for k in range(NVREG)
                )

            acc0 = tuple(jnp.zeros((L,), jnp.float32) for _ in range(NVREG))
            acc = lax.fori_loop(0, H, jbody, acc0)
            v = ((acc[0] + acc[1]) + (acc[2] + acc[3])) + (
                (acc[4] + acc[5]) + (acc[6] + acc[7]))
            lanes = lax.iota(jnp.int32, L)
            for sh in (1, 2, 4, 8):
                v = v + _lane_gather(v, lanes ^ sh)
            pos = CHUNK_ROWS * c + r
            plsc.store_scatter(
                out_v,
                [jnp.full((L,), pos, jnp.int32)],
                v,
                mask=lanes == 0,
            )
        return carry

    lax.fori_loop(0, N_CHUNKS, chunk_body, 0)
    pltpu.sync_copy(out_v, out.at[pl.ds(obase, ROWS_PER_W)])


@jax.jit
def _mf(bidx, sidx, btab, stab):
    mesh = plsc.VectorSubcoreMesh(core_axis_name="c", subcore_axis_name="s")
    return pl.kernel(
        _body,
        out_type=jax.ShapeDtypeStruct((B,), jnp.float32),
        mesh=mesh,
        scratch_types=[
            pltpu.VMEM((CHUNK_ROWS, H), jnp.int32),
            pltpu.VMEM((CHUNK_ROWS, H), jnp.int32),
            pltpu.VMEM((CHUNK_ROWS, H, F), jnp.float32),
            pltpu.VMEM((CHUNK_ROWS, H, F), jnp.float32),
            pltpu.VMEM((ROWS_PER_W,), jnp.float32),
            pltpu.SemaphoreType.DMA,
        ],
        compiler_params=pltpu.CompilerParams(
            needs_layout_passes=False, use_tc_tiling_on_sc=False),
    )(bidx, sidx, btab, stab)


def kernel(b, s, buyer_factors, seller_factors):
    return _mf(b, s, buyer_factors, seller_factors)


# speedup vs baseline: 18.7199x; 1.8843x over previous
import jax
import jax.numpy as jnp
from jax import lax
from jax.experimental import pallas as pl
from jax.experimental.pallas import tpu as pltpu
from jax.experimental.pallas import tpu_sc as plsc

B = 4096
H = 50
F = 128
NW = 32
ROWS_PER_W = B // NW
CHUNK_ROWS = 4
N_CHUNKS = ROWS_PER_W // CHUNK_ROWS
L = 16
NVREG = F // L


def _lane_gather(v, idx):
    dn = lax.GatherDimensionNumbers(
        offset_dims=(), collapsed_slice_dims=(0,), start_index_map=(0,))
    return lax.gather(v, idx[:, None], dn, slice_sizes=(1,),
                      mode=lax.GatherScatterMode.PROMISE_IN_BOUNDS)


def _body(bidx, sidx, btab, stab, out, idxb_v, idxs_v, brow_v, srow_v,
          out_v, sem0, sem1):
    wid = lax.axis_index("s") * 2 + lax.axis_index("c")
    obase = wid * ROWS_PER_W
    sems = (sem0, sem1)

    pltpu.sync_copy(bidx.at[pl.ds(obase, ROWS_PER_W)], idxb_v)
    pltpu.sync_copy(sidx.at[pl.ds(obase, ROWS_PER_W)], idxs_v)

    def fire(c, buf):
        sem = sems[buf]
        for r in range(CHUNK_ROWS):
            row = c * CHUNK_ROWS + r
            pltpu.async_copy(btab.at[idxb_v.at[row]], brow_v.at[buf, r], sem)
            pltpu.async_copy(stab.at[idxs_v.at[row]], srow_v.at[buf, r], sem)

    def drain(c, buf):
        sem = sems[buf]
        for r in range(CHUNK_ROWS):
            row = c * CHUNK_ROWS + r
            pltpu.make_async_copy(
                btab.at[idxb_v.at[row]], brow_v.at[buf, r], sem).wait()
            pltpu.make_async_copy(
                stab.at[idxs_v.at[row]], srow_v.at[buf, r], sem).wait()

    def compute(c, buf):
        for r in range(CHUNK_ROWS):
            def jbody(j, acc, r=r, buf=buf):
                return tuple(
                    acc[k]
                    + brow_v[buf, r, j, pl.ds(k * L, L)]
                    * srow_v[buf, r, j, pl.ds(k * L, L)]
                    for k in range(NVREG)
                )

            acc0 = tuple(jnp.zeros((L,), jnp.float32) for _ in range(NVREG))
            acc = lax.fori_loop(0, H, jbody, acc0, unroll=2)
            v = ((acc[0] + acc[1]) + (acc[2] + acc[3])) + (
                (acc[4] + acc[5]) + (acc[6] + acc[7]))
            lanes = lax.iota(jnp.int32, L)
            for sh in (1, 2, 4, 8):
                v = v + _lane_gather(v, lanes ^ sh)
            plsc.store_scatter(
                out_v,
                [jnp.full((L,), CHUNK_ROWS * c + r, jnp.int32)],
                v,
                mask=lanes == 0,
            )

    fire(0, 0)

    def step(t, carry):
        c0 = 2 * t
        c1 = c0 + 1
        fire(c1, 1)
        drain(c0, 0)
        compute(c0, 0)

        @pl.when(t + 1 < N_CHUNKS // 2)
        def _():
            fire(c0 + 2, 0)

        drain(c1, 1)
        compute(c1, 1)
        return carry

    lax.fori_loop(0, N_CHUNKS // 2, step, 0)
    pltpu.sync_copy(out_v, out.at[pl.ds(obase, ROWS_PER_W)])


@jax.jit
def _mf(bidx, sidx, btab, stab):
    mesh = plsc.VectorSubcoreMesh(core_axis_name="c", subcore_axis_name="s")
    return pl.kernel(
        _body,
        out_type=jax.ShapeDtypeStruct((B,), jnp.float32),
        mesh=mesh,
        scratch_types=[
            pltpu.VMEM((ROWS_PER_W, H), jnp.int32),
            pltpu.VMEM((ROWS_PER_W, H), jnp.int32),
            pltpu.VMEM((2, CHUNK_ROWS, H, F), jnp.float32),
            pltpu.VMEM((2, CHUNK_ROWS, H, F), jnp.float32),
            pltpu.VMEM((ROWS_PER_W,), jnp.float32),
            pltpu.SemaphoreType.DMA,
            pltpu.SemaphoreType.DMA,
        ],
        compiler_params=pltpu.CompilerParams(
            needs_layout_passes=False, use_tc_tiling_on_sc=False),
    )(bidx, sidx, btab, stab)


def kernel(b, s, buyer_factors, seller_factors):
    return _mf(b, s, buyer_factors, seller_factors)
